# baseline (speedup 1.0000x reference)
import functools

import jax
import jax.numpy as jnp
from jax import lax
from jax.experimental import pallas as pl
from jax.experimental.pallas import tpu as pltpu
from jax.experimental.pallas import tpu_sc as plsc

NC = 2
NS = 16
L = 16
H = 64
C = 40
ZC = 40
GOFF = (0, 16, 24)


def _node_tables_body(node_ref, wt_ref, wq_ref, tsrc_ref, tdst_ref, pd_ref):
    x = node_ref[...]
    wq = wq_ref[...]
    ps = jnp.dot(x, wt_ref[0:128, :], preferred_element_type=jnp.float32)
    pd = jnp.dot(x, wt_ref[256:384, :], preferred_element_type=jnp.float32)
    qs = jnp.dot(ps, wq, preferred_element_type=jnp.float32)
    qd = jnp.dot(pd, wq, preferred_element_type=jnp.float32)
    tsrc_ref[0, :, 0:H] = ps[:, 0:H]
    tsrc_ref[0, :, H:2 * H] = qs[:, 0:H]
    tsrc_ref[1, :, 0:H] = ps[:, H:2 * H]
    tsrc_ref[1, :, H:2 * H] = qs[:, H:2 * H]
    tdst_ref[...] = qd
    pd_ref[...] = pd


def _rel_tables_body(rel_ref, wt_ref, wq_ref, trel_ref, cs_ref):
    xr = rel_ref[...]
    wq = wq_ref[...]
    pr = jnp.dot(xr, wt_ref[128:256, :], preferred_element_type=jnp.float32)
    qr = jnp.dot(pr, wq, preferred_element_type=jnp.float32)
    trel_ref[0, :, 0:H] = pr[:, 0:H]
    trel_ref[0, :, H:2 * H] = qr[:, 0:H]
    trel_ref[1, :, 0:H] = pr[:, H:2 * H]
    trel_ref[1, :, H:2 * H] = qr[:, H:2 * H]
    cs_ref[...] = jnp.sum(wq, axis=0)


def _epilogue_body(a0_ref, a1_ref, pd_ref, norm_ref, o_ref):
    eps = 1e-16
    nrm = norm_ref[...]
    den0 = a0_ref[:, H:2 * H]
    den1 = a1_ref[:, H:2 * H]
    h0 = (a0_ref[:, 0:H] + pd_ref[:, 0:H] * den0) / (den0 + eps) * nrm
    h1 = (a1_ref[:, 0:H] + pd_ref[:, H:2 * H] * den1) / (den1 + eps) * nrm
    o_ref[:, 0:H] = jnp.maximum(h0, 0.0)
    o_ref[:, H:2 * H] = jnp.maximum(h1, 0.0)


def _make_sc_edge_kernel(n_nodes, n_rel, n_edges):
    e_sub = n_edges // NS
    n_chunks = e_sub // C
    K = 10
    NB = n_chunks // K
    nz = n_nodes // ZC
    nz_iters = (nz + NS - 1) // NS
    row_bytes = C * 2 * H * 4
    mesh = plsc.VectorSubcoreMesh(core_axis_name="c", subcore_axis_name="s")

    @functools.partial(
        pl.kernel,
        mesh=mesh,
        out_type=jax.ShapeDtypeStruct((NC * n_nodes, 2 * H), jnp.float32),
        scratch_types=[
            pltpu.VMEM((2 * K * C,), jnp.int32),
            pltpu.VMEM((2 * K * C,), jnp.int32),
            pltpu.VMEM((2 * K * C,), jnp.int32),
            pltpu.VMEM((2 * K * C,), jnp.float32),
            pltpu.VMEM((C,), jnp.int32),
            pltpu.VMEM((C,), jnp.int32),
            pltpu.VMEM((C, 2 * H), jnp.float32),
            pltpu.VMEM((C, 2 * H), jnp.float32),
            pltpu.VMEM((C, 2 * H), jnp.float32),
            pltpu.VMEM((C, 2 * H), jnp.float32),
            pltpu.VMEM((C, 2 * H), jnp.float32),
            pltpu.VMEM((C, 2 * H), jnp.float32),
            pltpu.VMEM((C, 2 * H), jnp.float32),
            pltpu.VMEM((C, 2 * H), jnp.float32),
            pltpu.VMEM((2 * H,), jnp.float32),
            pltpu.VMEM_SHARED((n_nodes, 2 * H), jnp.float32),
            pltpu.SemaphoreType.DMA,
            pltpu.SemaphoreType.DMA,
            pltpu.SemaphoreType.DMA,
        ],
    )
    def sc_edge(tsrc_hbm, trel_hbm, tdst_hbm, cs_hbm, src_hbm, et_hbm,
                dst_hbm, fre_hbm, o_hbm, sstage, estage, dstage, fstage,
                idxd2_0, idxd2_1, srows0, srows1, rrows0, rrows1,
                drows0, drows1, orows0, orows1, cs_v, acc_sh,
                isem, gsem, ssem):
        idxd2 = (idxd2_0, idxd2_1)
        srows = (srows0, srows1)
        rrows = (rrows0, rrows1)
        drows = (drows0, drows1)
        orows = (orows0, orows1)
        cid = lax.axis_index("c")
        sid = lax.axis_index("s")
        soff = cid * n_nodes
        roff = cid * n_rel
        e0 = sid * e_sub

        pltpu.sync_copy(cs_hbm.at[pl.ds(cid * H, H)], cs_v.at[pl.ds(0, H)])

        zeros16 = jnp.zeros((L,), jnp.float32)

        @pl.loop(0, C)
        def _(i):
            for k in range(2 * H // L):
                orows0[i, pl.ds(k * L, L)] = zeros16

        @pl.loop(0, nz_iters)
        def _(t):
            chunk = t * NS + sid

            @pl.when(chunk < nz)
            def _():
                pltpu.sync_copy(orows0, acc_sh.at[pl.ds(chunk * ZC, ZC)])

        plsc.subcore_barrier()

        def load_batch(boff, b):
            off = e0 + b * (K * C)
            pltpu.async_copy(src_hbm.at[pl.ds(off, K * C)],
                             sstage.at[pl.ds(boff, K * C)], isem)
            pltpu.async_copy(et_hbm.at[pl.ds(off, K * C)],
                             estage.at[pl.ds(boff, K * C)], isem)
            pltpu.async_copy(dst_hbm.at[pl.ds(off, K * C)],
                             dstage.at[pl.ds(boff, K * C)], isem)
            pltpu.async_copy(fre_hbm.at[pl.ds(off, K * C)],
                             fstage.at[pl.ds(boff, K * C)], isem)

        def wait_batch():
            for _ in range(4):
                pltpu.make_async_copy(src_hbm.at[pl.ds(e0, K * C)],
                                      sstage.at[pl.ds(0, K * C)], isem).wait()

        def adjust_batch(boff):
            @pl.loop(0, K * C // L)
            def _(t):
                sl = pl.ds(boff + t * L, L)
                sstage[sl] = sstage[sl] + soff
                estage[sl] = estage[sl] + roff

        def fire_gathers(sl, boff, p):
            pltpu.async_copy(tsrc_hbm.at[sstage.at[pl.ds(boff + p * C, C)]],
                             srows[sl], gsem)
            pltpu.async_copy(trel_hbm.at[estage.at[pl.ds(boff + p * C, C)]],
                             rrows[sl], gsem)
            pltpu.async_copy(tdst_hbm.at[dstage.at[pl.ds(boff + p * C, C)]],
                             drows[sl], gsem)

        def wait_gathers(sl):
            for _ in range(3):
                pltpu.make_async_copy(
                    tsrc_hbm.at[sstage.at[pl.ds(0, C)]],
                    srows[sl], gsem).wait()

        def drain_scatter():
            pltpu.make_async_copy(orows0, acc_sh.at[idxd2_0],
                                  ssem).wait()

        def compute_chunk(sl, boff, p):
            sv, rv, dv, ov = srows[sl], rrows[sl], drows[sl], orows[sl]
            csk = tuple(cs_v[pl.ds(k * L, L)] for k in range(H // L))
            for goff in GOFF:
                idxd2[sl][pl.ds(goff, L)] = dstage[pl.ds(boff + p * C
                                                         + goff, L)]

            @pl.loop(0, len(GOFF))
            def _(g):
                base = g * L - (g // 2) * 8
                fv = fstage[pl.ds(boff + p * C + base, L)]
                for e in range(L):
                    i = base + e
                    f = fv[e]
                    for k in range(H // L):
                        lo = pl.ds(k * L, L)
                        hi = pl.ds(H + k * L, L)
                        a = (sv[i, hi] + rv[i, hi]
                             + dv[i, pl.ds(cid * H + k * L, L)]
                             + f * csk[k])
                        a = jnp.maximum(a, 0.01 * a)
                        ex = jnp.exp(a)
                        ov[i, lo] = ex * (sv[i, lo] + rv[i, lo])
                        ov[i, hi] = ex

        def fire_scatter(sl):
            pltpu.async_copy(orows[sl], acc_sh.at[idxd2[sl]],
                             ssem, add=True)

        load_batch(0, 0)
        wait_batch()
        adjust_batch(0)
        fire_gathers(0, 0, 0)

        @pl.loop(0, NB)
        def _(b):
            boff = (b % 2) * (K * C)
            boff_n = ((b + 1) % 2) * (K * C)

            @pl.when(b > 0)
            def _():
                drain_scatter()
                drain_scatter()

            @pl.when(b + 1 < NB)
            def _():
                load_batch(boff_n, b + 1)

            @pl.loop(0, K // 2)
            def _(q):
                for s_ in range(2):
                    p = q * 2 + s_

                    @pl.when(q > 0)
                    def _():
                        drain_scatter()

                    wait_gathers(s_)

                    @pl.when(p + 1 < K)
                    def _():
                        fire_gathers(1 - s_, boff, p + 1)

                    compute_chunk(s_, boff, p)
                    fire_scatter(s_)

            @pl.when(b + 1 < NB)
            def _():
                wait_batch()
                adjust_batch(boff_n)
                fire_gathers(0, boff_n, 0)

        drain_scatter()
        drain_scatter()
        plsc.subcore_barrier()

        @pl.loop(0, nz_iters)
        def _(t):
            chunk = t * NS + sid

            @pl.when(chunk < nz)
            def _():
                pltpu.sync_copy(acc_sh.at[pl.ds(chunk * ZC, ZC)],
                                o_hbm.at[pl.ds(soff + chunk * ZC, ZC)])

    return sc_edge


def kernel(node, rel, edge_index, edge_type, fre, norm, w_triplet, w_quad):
    n_nodes, d = node.shape
    n_rel = rel.shape[0]
    n_edges = edge_type.shape[0]
    bn = 1000

    tsrc, tdst, pd = pl.pallas_call(
        _node_tables_body,
        grid=(n_nodes // bn,),
        in_specs=[
            pl.BlockSpec((bn, d), lambda i: (i, 0)),
            pl.BlockSpec((3 * d, d), lambda i: (0, 0)),
            pl.BlockSpec((d, d), lambda i: (0, 0)),
        ],
        out_specs=[
            pl.BlockSpec((NC, bn, d), lambda i: (0, i, 0)),
            pl.BlockSpec((bn, d), lambda i: (i, 0)),
            pl.BlockSpec((bn, d), lambda i: (i, 0)),
        ],
        out_shape=[
            jax.ShapeDtypeStruct((NC, n_nodes, d), jnp.float32),
            jax.ShapeDtypeStruct((n_nodes, d), jnp.float32),
            jax.ShapeDtypeStruct((n_nodes, d), jnp.float32),
        ],
    )(node, w_triplet, w_quad)

    trel, cs = pl.pallas_call(
        _rel_tables_body,
        out_shape=[
            jax.ShapeDtypeStruct((NC, n_rel, d), jnp.float32),
            jax.ShapeDtypeStruct((d,), jnp.float32),
        ],
    )(rel, w_triplet, w_quad)

    sc_edge = _make_sc_edge_kernel(n_nodes, n_rel, n_edges)
    acc = sc_edge(
        tsrc.reshape(NC * n_nodes, d),
        trel.reshape(NC * n_rel, d),
        tdst,
        cs,
        edge_index[0],
        edge_type,
        edge_index[1],
        fre,
    )

    out = pl.pallas_call(
        _epilogue_body,
        grid=(n_nodes // bn,),
        in_specs=[
            pl.BlockSpec((bn, d), lambda i: (i, 0)),
            pl.BlockSpec((bn, d), lambda i: (n_nodes // bn + i, 0)),
            pl.BlockSpec((bn, d), lambda i: (i, 0)),
            pl.BlockSpec((bn, 1), lambda i: (i, 0)),
        ],
        out_specs=pl.BlockSpec((bn, d), lambda i: (i, 0)),
        out_shape=jax.ShapeDtypeStruct((n_nodes, d), jnp.float32),
    )(acc, acc, pd, norm)

    return out


# speedup vs baseline: 1.0211x; 1.0211x over previous
import functools

import jax
import jax.numpy as jnp
from jax import lax
from jax.experimental import pallas as pl
from jax.experimental.pallas import tpu as pltpu
from jax.experimental.pallas import tpu_sc as plsc

NC = 2
NS = 16
L = 16
H = 64
C = 40
ZC = 40
GOFF = (0, 16, 24)


def _node_tables_body(node_ref, wt_ref, wq_ref, tsrc_ref, tdst_ref, pd_ref):
    x = node_ref[...]
    wq = wq_ref[...]
    ps = jnp.dot(x, wt_ref[0:128, :], preferred_element_type=jnp.float32)
    pd = jnp.dot(x, wt_ref[256:384, :], preferred_element_type=jnp.float32)
    qs = jnp.dot(ps, wq, preferred_element_type=jnp.float32)
    qd = jnp.dot(pd, wq, preferred_element_type=jnp.float32)
    tsrc_ref[0, :, 0:H] = ps[:, 0:H]
    tsrc_ref[0, :, H:2 * H] = qs[:, 0:H]
    tsrc_ref[1, :, 0:H] = ps[:, H:2 * H]
    tsrc_ref[1, :, H:2 * H] = qs[:, H:2 * H]
    tdst_ref[...] = qd
    pd_ref[...] = pd


def _rel_tables_body(rel_ref, wt_ref, wq_ref, trel_ref, cs_ref):
    xr = rel_ref[...]
    wq = wq_ref[...]
    pr = jnp.dot(xr, wt_ref[128:256, :], preferred_element_type=jnp.float32)
    qr = jnp.dot(pr, wq, preferred_element_type=jnp.float32)
    trel_ref[0, :, 0:H] = pr[:, 0:H]
    trel_ref[0, :, H:2 * H] = qr[:, 0:H]
    trel_ref[1, :, 0:H] = pr[:, H:2 * H]
    trel_ref[1, :, H:2 * H] = qr[:, H:2 * H]
    cs_ref[...] = jnp.sum(wq, axis=0)


def _epilogue_body(a0_ref, a1_ref, pd_ref, norm_ref, o_ref):
    eps = 1e-16
    nrm = norm_ref[...]
    den0 = a0_ref[:, H:2 * H]
    den1 = a1_ref[:, H:2 * H]
    h0 = (a0_ref[:, 0:H] + pd_ref[:, 0:H] * den0) / (den0 + eps) * nrm
    h1 = (a1_ref[:, 0:H] + pd_ref[:, H:2 * H] * den1) / (den1 + eps) * nrm
    o_ref[:, 0:H] = jnp.maximum(h0, 0.0)
    o_ref[:, H:2 * H] = jnp.maximum(h1, 0.0)


def _make_sc_edge_kernel(n_nodes, n_rel, n_edges):
    e_sub = n_edges // NS
    n_chunks = e_sub // C
    K = 10
    NB = n_chunks // K
    CI = 3 * C
    BI = K * CI
    dbase = 2 * n_nodes + 2 * n_rel
    nz = n_nodes // ZC
    nz_iters = (nz + NS - 1) // NS
    mesh = plsc.VectorSubcoreMesh(core_axis_name="c", subcore_axis_name="s")

    @functools.partial(
        pl.kernel,
        mesh=mesh,
        out_type=jax.ShapeDtypeStruct((NC * n_nodes, 2 * H), jnp.float32),
        scratch_types=[
            pltpu.VMEM((2 * BI,), jnp.int32),
            pltpu.VMEM((2 * K * C,), jnp.float32),
            pltpu.VMEM((BI,), jnp.int32),
            pltpu.VMEM((C,), jnp.int32),
            pltpu.VMEM((C,), jnp.int32),
            pltpu.VMEM((CI, 2 * H), jnp.float32),
            pltpu.VMEM((CI, 2 * H), jnp.float32),
            pltpu.VMEM((C, 2 * H), jnp.float32),
            pltpu.VMEM((C, 2 * H), jnp.float32),
            pltpu.VMEM((2 * H,), jnp.float32),
            pltpu.VMEM_SHARED((n_nodes, 2 * H), jnp.float32),
            pltpu.SemaphoreType.DMA,
            pltpu.SemaphoreType.DMA,
            pltpu.SemaphoreType.DMA,
        ],
    )
    def sc_edge(tcomb_hbm, cs_hbm, ci_hbm, fre_hbm, pat_hbm, o_hbm,
                cstage, fstage, patv, idxd2_0, idxd2_1, crows0, crows1,
                orows0, orows1, cs_v, acc_sh, isem, gsem, ssem):
        idxd2 = (idxd2_0, idxd2_1)
        crows = (crows0, crows1)
        orows = (orows0, orows1)
        cid = lax.axis_index("c")
        sid = lax.axis_index("s")
        e0c = sid * (n_chunks * CI)
        e0f = sid * e_sub

        pltpu.sync_copy(cs_hbm.at[pl.ds(cid * H, H)], cs_v.at[pl.ds(0, H)])
        pltpu.sync_copy(pat_hbm, patv)

        @pl.loop(0, BI // L)
        def _(t):
            sl = pl.ds(t * L, L)
            patv[sl] = patv[sl] * cid

        zeros16 = jnp.zeros((L,), jnp.float32)

        @pl.loop(0, C)
        def _(i):
            for k in range(2 * H // L):
                orows0[i, pl.ds(k * L, L)] = zeros16

        @pl.loop(0, nz_iters)
        def _(t):
            chunk = t * NS + sid

            @pl.when(chunk < nz)
            def _():
                pltpu.sync_copy(orows0, acc_sh.at[pl.ds(chunk * ZC, ZC)])

        plsc.subcore_barrier()

        def load_batch(bs, b):
            pltpu.async_copy(ci_hbm.at[pl.ds(e0c + b * BI, BI)],
                             cstage.at[pl.ds(bs * BI, BI)], isem)
            pltpu.async_copy(fre_hbm.at[pl.ds(e0f + b * (K * C), K * C)],
                             fstage.at[pl.ds(bs * (K * C), K * C)], isem)

        def wait_batch():
            pltpu.make_async_copy(ci_hbm.at[pl.ds(e0c, BI)],
                                  cstage.at[pl.ds(0, BI)], isem).wait()
            pltpu.make_async_copy(fre_hbm.at[pl.ds(e0f, K * C)],
                                  fstage.at[pl.ds(0, K * C)], isem).wait()

        def adjust_batch(bs):
            @pl.loop(0, BI // L)
            def _(t):
                sl = pl.ds(bs * BI + t * L, L)
                cstage[sl] = cstage[sl] + patv[pl.ds(t * L, L)]

        def fire_gathers(sl, bs, p):
            pltpu.async_copy(
                tcomb_hbm.at[cstage.at[pl.ds(bs * BI + p * CI, CI)]],
                crows[sl], gsem)

        def wait_gathers(sl):
            pltpu.make_async_copy(tcomb_hbm.at[cstage.at[pl.ds(0, CI)]],
                                  crows[sl], gsem).wait()

        def drain_scatter():
            pltpu.make_async_copy(orows0, acc_sh.at[idxd2_0],
                                  ssem).wait()

        def compute_chunk(sl, bs, p):
            cv, ov = crows[sl], orows[sl]
            coff = bs * BI + p * CI
            foff = bs * (K * C) + p * C
            csk = tuple(cs_v[pl.ds(k * L, L)] for k in range(H // L))
            for goff in GOFF:
                idxd2[sl][pl.ds(goff, L)] = (cstage[pl.ds(coff + 2 * C
                                                          + goff, L)]
                                             - dbase)

            @pl.loop(0, len(GOFF))
            def _(g):
                base = g * L - (g // 2) * 8
                fv = fstage[pl.ds(foff + base, L)]
                for e in range(L):
                    i = base + e
                    f = fv[e]
                    for k in range(H // L):
                        lo = pl.ds(k * L, L)
                        hi = pl.ds(H + k * L, L)
                        a = (cv[i, hi] + cv[C + i, hi]
                             + cv[2 * C + i, pl.ds(cid * H + k * L, L)]
                             + f * csk[k])
                        a = jnp.maximum(a, 0.01 * a)
                        ex = jnp.exp(a)
                        ov[i, lo] = ex * (cv[i, lo] + cv[C + i, lo])
                        ov[i, hi] = ex

        def fire_scatter(sl):
            pltpu.async_copy(orows[sl], acc_sh.at[idxd2[sl]],
                             ssem, add=True)

        load_batch(0, 0)
        wait_batch()
        adjust_batch(0)
        fire_gathers(0, 0, 0)

        @pl.loop(0, NB)
        def _(b):
            bs = b % 2
            bs_n = (b + 1) % 2

            @pl.when(b > 0)
            def _():
                drain_scatter()
                drain_scatter()

            @pl.when(b + 1 < NB)
            def _():
                load_batch(bs_n, b + 1)

            @pl.loop(0, K // 2)
            def _(q):
                for s_ in range(2):
                    p = q * 2 + s_

                    @pl.when(q > 0)
                    def _():
                        drain_scatter()

                    wait_gathers(s_)

                    @pl.when(p + 1 < K)
                    def _():
                        fire_gathers(1 - s_, bs, p + 1)

                    compute_chunk(s_, bs, p)
                    fire_scatter(s_)

            @pl.when(b + 1 < NB)
            def _():
                wait_batch()
                adjust_batch(bs_n)
                fire_gathers(0, bs_n, 0)

        drain_scatter()
        drain_scatter()
        plsc.subcore_barrier()

        @pl.loop(0, nz_iters)
        def _(t):
            chunk = t * NS + sid

            @pl.when(chunk < nz)
            def _():
                pltpu.sync_copy(acc_sh.at[pl.ds(chunk * ZC, ZC)],
                                o_hbm.at[pl.ds(cid * n_nodes + chunk * ZC,
                                               ZC)])

    return sc_edge


def kernel(node, rel, edge_index, edge_type, fre, norm, w_triplet, w_quad):
    n_nodes, d = node.shape
    n_rel = rel.shape[0]
    n_edges = edge_type.shape[0]
    bn = 1000

    tsrc, tdst, pd = pl.pallas_call(
        _node_tables_body,
        grid=(n_nodes // bn,),
        in_specs=[
            pl.BlockSpec((bn, d), lambda i: (i, 0)),
            pl.BlockSpec((3 * d, d), lambda i: (0, 0)),
            pl.BlockSpec((d, d), lambda i: (0, 0)),
        ],
        out_specs=[
            pl.BlockSpec((NC, bn, d), lambda i: (0, i, 0)),
            pl.BlockSpec((bn, d), lambda i: (i, 0)),
            pl.BlockSpec((bn, d), lambda i: (i, 0)),
        ],
        out_shape=[
            jax.ShapeDtypeStruct((NC, n_nodes, d), jnp.float32),
            jax.ShapeDtypeStruct((n_nodes, d), jnp.float32),
            jax.ShapeDtypeStruct((n_nodes, d), jnp.float32),
        ],
    )(node, w_triplet, w_quad)

    trel, cs = pl.pallas_call(
        _rel_tables_body,
        out_shape=[
            jax.ShapeDtypeStruct((NC, n_rel, d), jnp.float32),
            jax.ShapeDtypeStruct((d,), jnp.float32),
        ],
    )(rel, w_triplet, w_quad)

    tcomb = jnp.concatenate(
        [tsrc.reshape(NC * n_nodes, d), trel.reshape(NC * n_rel, d), tdst],
        axis=0)
    ci = jnp.concatenate(
        [edge_index[0].reshape(-1, C),
         edge_type.reshape(-1, C) + NC * n_nodes,
         edge_index[1].reshape(-1, C) + NC * n_nodes + NC * n_rel],
        axis=1).reshape(-1)
    pat = jnp.tile(
        jnp.concatenate([jnp.full((C,), n_nodes, jnp.int32),
                         jnp.full((C,), n_rel, jnp.int32),
                         jnp.zeros((C,), jnp.int32)]), 10)
    sc_edge = _make_sc_edge_kernel(n_nodes, n_rel, n_edges)
    acc = sc_edge(tcomb, cs, ci, fre, pat)

    out = pl.pallas_call(
        _epilogue_body,
        grid=(n_nodes // bn,),
        in_specs=[
            pl.BlockSpec((bn, d), lambda i: (i, 0)),
            pl.BlockSpec((bn, d), lambda i: (n_nodes // bn + i, 0)),
            pl.BlockSpec((bn, d), lambda i: (i, 0)),
            pl.BlockSpec((bn, 1), lambda i: (i, 0)),
        ],
        out_specs=pl.BlockSpec((bn, d), lambda i: (i, 0)),
        out_shape=jax.ShapeDtypeStruct((n_nodes, d), jnp.float32),
    )(acc, acc, pd, norm)

    return out
